# baseline (speedup 1.0000x reference)
import functools

import jax
import jax.numpy as jnp
from jax import lax
from jax.experimental import pallas as pl
from jax.experimental.pallas import tpu as pltpu
from jax.experimental.pallas import tpu_sc as plsc

_B = 1024
_S = 200
_D = 64
_NTIME = 512
_NACT = 8
_NTOK = 4
_N = _B * _S
_NC = 2
_NS = 16
_NW = _NC * _NS
_PW = _N // _NW
_BLK = 128
_NB = _PW // _BLK
_UPW = _B // _NW

_BBLK = 32
_GRID = _B // _BBLK
_RB = _BBLK * _S


_C = 640
_NCH = _PW // _C


def _sc_gather_body(item_t, if0h, if1h, comb_t, user_t, uf0t, uf1t, uf2t,
                    tok_id, tok_f0, tok_f1,
                    pos_id, pos_f0, pos_f1,
                    neg_id, neg_f0, neg_f1,
                    comb_i, uid, u0, u1, u2,
                    tok_g, pos_g, neg_g, comb_g, user_g,
                    ia0, ia1, ia2, ib0, ib1, ib2, buf_a, buf_b, uidx,
                    sga, saa, swa, sgb, sab, swb):
  wid = lax.axis_index("s") * _NC + lax.axis_index("c")
  row0 = wid * _PW

  def start_g(tab, idxv, c, buf, sem, add=False):
    pltpu.async_copy(tab.at[idxv.at[c]], buf, sem, add=add)

  def wait_g(tab, idxv, c, buf, sem):
    pltpu.make_async_copy(tab.at[idxv.at[c]], buf, sem).wait()

  def start_w(buf, out, c, sem):
    pltpu.async_copy(buf, out.at[pl.ds(row0 + c * _C, _C)], sem)

  def wait_w(buf, out, c, sem):
    pltpu.make_async_copy(buf, out.at[pl.ds(row0 + c * _C, _C)], sem).wait()

  def phase(ta0, ta1, ta2, oa, tb0, tb1, tb2, ob):
    b_adds = tb1 is not None
    start_g(ta0, ia0, 0, buf_a, sga)
    start_g(tb0, ib0, 0, buf_b, sgb)

    def body(c, carry):
      wait_g(ta0, ia0, c, buf_a, sga)
      start_g(ta1, ia1, c, buf_a, saa, add=True)
      start_g(ta2, ia2, c, buf_a, saa, add=True)
      wait_g(tb0, ib0, c, buf_b, sgb)
      if b_adds:
        start_g(tb1, ib1, c, buf_b, sab, add=True)
        start_g(tb2, ib2, c, buf_b, sab, add=True)
      wait_g(ta1, ia1, c, buf_a, saa)
      wait_g(ta2, ia2, c, buf_a, saa)
      start_w(buf_a, oa, c, swa)
      if b_adds:
        wait_g(tb1, ib1, c, buf_b, sab)
        wait_g(tb2, ib2, c, buf_b, sab)
      start_w(buf_b, ob, c, swb)
      wait_w(buf_a, oa, c, swa)
      wait_w(buf_b, ob, c, swb)

      @pl.when(c + 1 < _NCH)
      def _():
        start_g(ta0, ia0, c + 1, buf_a, sga)
        start_g(tb0, ib0, c + 1, buf_b, sgb)

      return carry

    lax.fori_loop(0, _NCH, body, 0)

  pltpu.sync_copy(tok_id.at[wid], ia0)
  pltpu.sync_copy(tok_f0.at[wid], ia1)
  pltpu.sync_copy(tok_f1.at[wid], ia2)
  pltpu.sync_copy(pos_id.at[wid], ib0)
  pltpu.sync_copy(pos_f0.at[wid], ib1)
  pltpu.sync_copy(pos_f1.at[wid], ib2)
  phase(item_t, if0h, if1h, tok_g, item_t, if0h, if1h, pos_g)

  pltpu.sync_copy(neg_id.at[wid], ia0)
  pltpu.sync_copy(neg_f0.at[wid], ia1)
  pltpu.sync_copy(neg_f1.at[wid], ia2)
  pltpu.sync_copy(comb_i.at[wid], ib0)
  phase(item_t, if0h, if1h, neg_g, comb_t, None, None, comb_g)

  pltpu.sync_copy(uid.at[wid], uidx.at[0])
  pltpu.sync_copy(u0.at[wid], uidx.at[1])
  pltpu.sync_copy(u1.at[wid], uidx.at[2])
  pltpu.sync_copy(u2.at[wid], uidx.at[3])
  urows = buf_a.at[pl.ds(0, _UPW)]
  pltpu.async_copy(user_t.at[uidx.at[0]], urows, sga).wait()
  c1 = pltpu.async_copy(uf0t.at[uidx.at[1]], urows, sga, add=True)
  c2 = pltpu.async_copy(uf1t.at[uidx.at[2]], urows, saa, add=True)
  c3 = pltpu.async_copy(uf2t.at[uidx.at[3]], urows, swa, add=True)
  c1.wait()
  c2.wait()
  c3.wait()
  pltpu.sync_copy(urows, user_g.at[pl.ds(wid * _UPW, _UPW)])


def _sc_gather(item_t, if0h, if1h, comb_t, user_t, uf0t, uf1t, uf2t,
               tok_id, tok_f0, tok_f1, pos_id, pos_f0, pos_f1,
               neg_id, neg_f0, neg_f1, comb_i, uid, u0, u1, u2):
  mesh = plsc.VectorSubcoreMesh(core_axis_name="c", subcore_axis_name="s")
  f32 = jnp.float32
  out_type = (
      jax.ShapeDtypeStruct((_N, _D), f32),
      jax.ShapeDtypeStruct((_N, _D), f32),
      jax.ShapeDtypeStruct((_N, _D), f32),
      jax.ShapeDtypeStruct((_N, _D), f32),
      jax.ShapeDtypeStruct((_B, _D), f32),
  )
  scratch = [
      pltpu.VMEM((_NCH, _C), jnp.int32),
      pltpu.VMEM((_NCH, _C), jnp.int32),
      pltpu.VMEM((_NCH, _C), jnp.int32),
      pltpu.VMEM((_NCH, _C), jnp.int32),
      pltpu.VMEM((_NCH, _C), jnp.int32),
      pltpu.VMEM((_NCH, _C), jnp.int32),
      pltpu.VMEM((_C, _D), f32),
      pltpu.VMEM((_C, _D), f32),
      pltpu.VMEM((4, _UPW), jnp.int32),
      pltpu.SemaphoreType.DMA,
      pltpu.SemaphoreType.DMA,
      pltpu.SemaphoreType.DMA,
      pltpu.SemaphoreType.DMA,
      pltpu.SemaphoreType.DMA,
      pltpu.SemaphoreType.DMA,
  ]
  run = pl.kernel(_sc_gather_body, out_type=out_type, mesh=mesh,
                  scratch_types=scratch,
                  compiler_params=pltpu.CompilerParams(
                      use_tc_tiling_on_sc=False))
  return run(item_t, if0h, if1h, comb_t, user_t, uf0t, uf1t, uf2t,
             tok_id, tok_f0, tok_f1, pos_id, pos_f0, pos_f1,
             neg_id, neg_f0, neg_f1, comb_i, uid, u0, u1, u2)


def _tc_body(tok_ref, pos_ref, neg_ref, comb_ref, user_ref, sel_ref, tt_ref,
             wi_ref, bi_ref, wu_ref, bu_ref, g_ref, be_ref, out_ref, acc):
  i = pl.program_id(0)
  f32 = jnp.float32

  wi = wi_ref[...]
  bi = bi_ref[...]

  def mm_tanh(ref, w, b):
    out = jnp.tanh(jnp.dot(ref[...], w, preferred_element_type=f32) + b)
    return out.reshape(_BBLK, _S, _D)

  tokens = mm_tanh(tok_ref, wi, bi)

  user_token = jnp.tanh(
      jnp.dot(user_ref[...], wu_ref[...], preferred_element_type=f32)
      + bu_ref[...])
  usr = jnp.broadcast_to(user_token[:, None, :], (_BBLK, _S, _D))
  sel = jnp.broadcast_to(sel_ref[...][:, :, None], (_BBLK, _S, _D))
  tokens = jnp.where(sel > 0, usr, tokens)

  x = tokens + comb_ref[...].reshape(_BBLK, _S, _D)
  mu = jnp.mean(x, axis=2, keepdims=True)
  xc = x - jnp.broadcast_to(mu, (_BBLK, _S, _D))
  var = jnp.mean(xc * xc, axis=2, keepdims=True)
  xn = (g_ref[...][None] * xc
        * jnp.broadcast_to(lax.rsqrt(var + 1e-5), (_BBLK, _S, _D))
        + be_ref[...][None])

  pos_t = mm_tanh(pos_ref, wi, bi)
  neg_t = mm_tanh(neg_ref, wi, bi)

  pos_l = jnp.sum(xn * pos_t, axis=2, keepdims=True)
  neg_l = jnp.sum(xn * neg_t, axis=2, keepdims=True)
  m = (tt_ref[...][:, :, None] == 1).astype(f32)

  def softplus(v):
    return jnp.maximum(v, 0.0) + jnp.log1p(jnp.exp(-jnp.abs(v)))

  blk_m = jnp.sum(m)
  blk_p = jnp.sum(pos_l * m)
  blk_n = jnp.sum(neg_l * m)
  blk_n2 = jnp.sum(neg_l * neg_l * m)
  blk_l = jnp.sum((softplus(-pos_l) + softplus(neg_l)) * m)
  blk_mx = jnp.max(jnp.where(m > 0, neg_l, -jnp.inf))

  @pl.when(i == 0)
  def _():
    acc[0] = 0.0
    acc[1] = 0.0
    acc[2] = 0.0
    acc[3] = 0.0
    acc[4] = 0.0
    acc[5] = -jnp.inf

  acc[0] = acc[0] + blk_m
  acc[1] = acc[1] + blk_p
  acc[2] = acc[2] + blk_n
  acc[3] = acc[3] + blk_n2
  acc[4] = acc[4] + blk_l
  acc[5] = jnp.maximum(acc[5], blk_mx)

  @pl.when(i == _GRID - 1)
  def _():
    denom = acc[0] + 1e-8
    neg_score = acc[2] / denom
    out_ref[0] = acc[4] / denom
    out_ref[1] = acc[1] / denom
    out_ref[2] = neg_score
    out_ref[3] = (acc[3] - 2.0 * neg_score * acc[2]
                  + neg_score * neg_score * acc[0]) / denom
    out_ref[4] = acc[5]
    out_ref[5] = 0.0
    out_ref[6] = 0.0
    out_ref[7] = 0.0


def _tc_reduce(tok_g, pos_g, neg_g, comb_g, user_g, sel, tt,
               W_item, b_item, W_user, b_user, ln_g, ln_b):
  f32 = jnp.float32
  val3 = pl.BlockSpec((_RB, _D), lambda i: (i, 0))
  msk2 = pl.BlockSpec((_BBLK, _S), lambda i: (i, 0))
  mat = pl.BlockSpec((_D, _D), lambda i: (0, 0))
  vec = pl.BlockSpec((1, _D), lambda i: (0, 0))
  return pl.pallas_call(
      _tc_body,
      grid=(_GRID,),
      in_specs=[
          val3, val3, val3, val3,
          pl.BlockSpec((_BBLK, _D), lambda i: (i, 0)),
          msk2, msk2,
          mat, vec, mat, vec, vec, vec,
      ],
      out_specs=pl.BlockSpec((8,), lambda i: (0,), memory_space=pltpu.SMEM),
      out_shape=jax.ShapeDtypeStruct((8,), f32),
      scratch_shapes=[pltpu.SMEM((8,), f32)],
  )(tok_g, pos_g, neg_g, comb_g, user_g, sel, tt,
    W_item, b_item, W_user, b_user, ln_g, ln_b)


def kernel(user_id, j, user_feat, id_seq, feat_seq, pos_seq, pos_feat,
           neg_seq, neg_feat, inter_time, act_type, token_type,
           user_table, item_table, uf0, uf1, uf2, if0, if1,
           W_user, b_user, W_item, b_item, time_table, act_table, tok_table,
           ln_g, ln_b):
  i32 = jnp.int32
  f32 = jnp.float32

  def shard(a):
    return a.astype(i32).reshape(_NW, _NCH, _C)

  tok_id = shard(id_seq)
  tok_f0 = shard(feat_seq[..., 0])
  tok_f1 = shard(feat_seq[..., 1])
  pos_id = shard(pos_seq)
  pos_f0 = shard(pos_feat[..., 0])
  pos_f1 = shard(pos_feat[..., 1])
  neg_id = shard(neg_seq)
  neg_f0 = shard(neg_feat[..., 0])
  neg_f1 = shard(neg_feat[..., 1])
  comb_i = shard((inter_time % _NTIME) * (_NACT * _NTOK)
                 + act_type * _NTOK + token_type)
  uid = user_id.astype(i32).reshape(_NW, _UPW)
  u0 = user_feat[:, 0].astype(i32).reshape(_NW, _UPW)
  u1 = user_feat[:, 1].astype(i32).reshape(_NW, _UPW)
  u2 = user_feat[:, 2].astype(i32).reshape(_NW, _UPW)

  if0h = if0 * 0.5
  if1h = if1 * 0.5
  third = f32(1.0 / 3.0)
  uf0t = uf0 * third
  uf1t = uf1 * third
  uf2t = uf2 * third
  comb_t = (time_table[:, None, None, :] + act_table[None, :, None, :]
            + tok_table[None, None, :, :]).reshape(_NTIME * _NACT * _NTOK, _D)

  tok_g, pos_g, neg_g, comb_g, user_g = _sc_gather(
      item_table, if0h, if1h, comb_t, user_table, uf0t, uf1t, uf2t,
      tok_id, tok_f0, tok_f1, pos_id, pos_f0, pos_f1,
      neg_id, neg_f0, neg_f1, comb_i, uid, u0, u1, u2)

  jj = jnp.clip(j, 0, _S - 1).astype(i32)
  sel = (jnp.arange(_S, dtype=i32)[None, :] == jj[:, None]).astype(i32)
  tt = token_type.astype(i32)

  out = _tc_reduce(
      tok_g, pos_g, neg_g, comb_g, user_g, sel, tt,
      W_item.astype(f32), b_item.reshape(1, _D).astype(f32),
      W_user.astype(f32), b_user.reshape(1, _D).astype(f32),
      ln_g.reshape(1, _D).astype(f32), ln_b.reshape(1, _D).astype(f32))

  return (out[0], out[1], out[2], out[3], out[4])


# speedup vs baseline: 1.3028x; 1.3028x over previous
import functools

import jax
import jax.numpy as jnp
from jax import lax
from jax.experimental import pallas as pl
from jax.experimental.pallas import tpu as pltpu
from jax.experimental.pallas import tpu_sc as plsc

_B = 1024
_S = 200
_D = 64
_NTIME = 512
_NACT = 8
_NTOK = 4
_N = _B * _S
_NC = 2
_NS = 16
_NW = _NC * _NS
_PW = _N // _NW
_BLK = 128
_NB = _PW // _BLK
_UPW = _B // _NW

_BBLK = 32
_GRID = _B // _BBLK
_RB = _BBLK * _S


_C = 640
_NCH = _PW // _C


def _sc_gather_body(item_t, if0h, if1h, comb_t, user_t, uf0t, uf1t, uf2t,
                    tok_id, tok_f0, tok_f1,
                    pos_id, pos_f0, pos_f1,
                    neg_id, neg_f0, neg_f1,
                    comb_i, uid, u0, u1, u2,
                    tok_g, pos_g, neg_g, comb_g, user_g,
                    ia0, ia1, ia2, ib0, ib1, ib2, buf_a, buf_b, uidx,
                    sga, saa, swa, sgb, sab, swb):
  wid = lax.axis_index("s") * _NC + lax.axis_index("c")
  row0 = wid * _PW

  def start_g(tab, idxv, c, buf, sem, add=False):
    pltpu.async_copy(tab.at[idxv.at[c]], buf, sem, add=add)

  def wait_g(tab, idxv, c, buf, sem):
    pltpu.make_async_copy(tab.at[idxv.at[c]], buf, sem).wait()

  def start_w(buf, out, c, sem):
    pltpu.async_copy(buf, out.at[pl.ds(row0 + c * _C, _C)], sem)

  def wait_w(buf, out, c, sem):
    pltpu.make_async_copy(buf, out.at[pl.ds(row0 + c * _C, _C)], sem).wait()

  def phase(ta0, ta1, ta2, oa, tb0, tb1, tb2, ob):
    b_adds = tb1 is not None
    start_g(ta0, ia0, 0, buf_a, sga)
    start_g(tb0, ib0, 0, buf_b, sgb)

    def body(c, carry):
      wait_g(ta0, ia0, c, buf_a, sga)
      start_g(ta1, ia1, c, buf_a, saa, add=True)
      start_g(ta2, ia2, c, buf_a, saa, add=True)
      wait_g(tb0, ib0, c, buf_b, sgb)
      if b_adds:
        start_g(tb1, ib1, c, buf_b, sab, add=True)
        start_g(tb2, ib2, c, buf_b, sab, add=True)
      wait_g(ta1, ia1, c, buf_a, saa)
      wait_g(ta2, ia2, c, buf_a, saa)
      start_w(buf_a, oa, c, swa)
      if b_adds:
        wait_g(tb1, ib1, c, buf_b, sab)
        wait_g(tb2, ib2, c, buf_b, sab)
      start_w(buf_b, ob, c, swb)
      wait_w(buf_a, oa, c, swa)
      wait_w(buf_b, ob, c, swb)

      @pl.when(c + 1 < _NCH)
      def _():
        start_g(ta0, ia0, c + 1, buf_a, sga)
        start_g(tb0, ib0, c + 1, buf_b, sgb)

      return carry

    lax.fori_loop(0, _NCH, body, 0)

  pltpu.sync_copy(tok_id.at[wid], ia0)
  pltpu.sync_copy(tok_f0.at[wid], ia1)
  pltpu.sync_copy(tok_f1.at[wid], ia2)
  pltpu.sync_copy(pos_id.at[wid], ib0)
  pltpu.sync_copy(pos_f0.at[wid], ib1)
  pltpu.sync_copy(pos_f1.at[wid], ib2)
  phase(item_t, if0h, if1h, tok_g, item_t, if0h, if1h, pos_g)

  pltpu.sync_copy(neg_id.at[wid], ia0)
  pltpu.sync_copy(neg_f0.at[wid], ia1)
  pltpu.sync_copy(neg_f1.at[wid], ia2)
  pltpu.sync_copy(comb_i.at[wid], ib0)
  phase(item_t, if0h, if1h, neg_g, comb_t, None, None, comb_g)

  pltpu.sync_copy(uid.at[wid], uidx.at[0])
  pltpu.sync_copy(u0.at[wid], uidx.at[1])
  pltpu.sync_copy(u1.at[wid], uidx.at[2])
  pltpu.sync_copy(u2.at[wid], uidx.at[3])
  urows = buf_a.at[pl.ds(0, _UPW)]
  pltpu.async_copy(user_t.at[uidx.at[0]], urows, sga).wait()
  c1 = pltpu.async_copy(uf0t.at[uidx.at[1]], urows, sga, add=True)
  c2 = pltpu.async_copy(uf1t.at[uidx.at[2]], urows, saa, add=True)
  c3 = pltpu.async_copy(uf2t.at[uidx.at[3]], urows, swa, add=True)
  c1.wait()
  c2.wait()
  c3.wait()
  pltpu.sync_copy(urows, user_g.at[pl.ds(wid * _UPW, _UPW)])


def _sc_gather(item_t, if0h, if1h, comb_t, user_t, uf0t, uf1t, uf2t,
               tok_id, tok_f0, tok_f1, pos_id, pos_f0, pos_f1,
               neg_id, neg_f0, neg_f1, comb_i, uid, u0, u1, u2):
  mesh = plsc.VectorSubcoreMesh(core_axis_name="c", subcore_axis_name="s")
  f32 = jnp.float32
  out_type = (
      jax.ShapeDtypeStruct((_N, _D), f32),
      jax.ShapeDtypeStruct((_N, _D), f32),
      jax.ShapeDtypeStruct((_N, _D), f32),
      jax.ShapeDtypeStruct((_N, _D), f32),
      jax.ShapeDtypeStruct((_B, _D), f32),
  )
  scratch = [
      pltpu.VMEM((_NCH, _C), jnp.int32),
      pltpu.VMEM((_NCH, _C), jnp.int32),
      pltpu.VMEM((_NCH, _C), jnp.int32),
      pltpu.VMEM((_NCH, _C), jnp.int32),
      pltpu.VMEM((_NCH, _C), jnp.int32),
      pltpu.VMEM((_NCH, _C), jnp.int32),
      pltpu.VMEM((_C, _D), f32),
      pltpu.VMEM((_C, _D), f32),
      pltpu.VMEM((4, _UPW), jnp.int32),
      pltpu.SemaphoreType.DMA,
      pltpu.SemaphoreType.DMA,
      pltpu.SemaphoreType.DMA,
      pltpu.SemaphoreType.DMA,
      pltpu.SemaphoreType.DMA,
      pltpu.SemaphoreType.DMA,
  ]
  run = pl.kernel(_sc_gather_body, out_type=out_type, mesh=mesh,
                  scratch_types=scratch,
                  compiler_params=pltpu.CompilerParams(
                      use_tc_tiling_on_sc=False))
  return run(item_t, if0h, if1h, comb_t, user_t, uf0t, uf1t, uf2t,
             tok_id, tok_f0, tok_f1, pos_id, pos_f0, pos_f1,
             neg_id, neg_f0, neg_f1, comb_i, uid, u0, u1, u2)


_S2 = _S // 2
_RB2 = _BBLK * _S2


def _tc_body(tok_ref, pos_ref, neg_ref, comb_ref, user_ref,
             sel_e_ref, sel_o_ref, m_e_ref, m_o_ref,
             wi_ref, bi_ref, wu_ref, bu_ref, g_ref, be_ref, out_ref, acc):
  i = pl.program_id(0)
  f32 = jnp.float32

  wi = wi_ref[...]
  bi = bi_ref[...]

  def mm_tanh(ref):
    out = jnp.tanh(jnp.dot(ref[...], wi, preferred_element_type=f32) + bi)
    return out.reshape(_BBLK, _S2, 128)

  def halves(v):
    return v[..., :_D], v[..., _D:]

  tok_e, tok_o = halves(mm_tanh(tok_ref))

  user_token = jnp.tanh(
      jnp.dot(user_ref[...], wu_ref[...], preferred_element_type=f32)
      + bu_ref[...])
  usr = jnp.broadcast_to(user_token[:, None, :], (_BBLK, _S2, _D))

  def overwrite(t, sel_ref_):
    sel = jnp.broadcast_to(sel_ref_[...][:, :, None], (_BBLK, _S2, _D))
    return jnp.where(sel > 0, usr, t)

  tok_e = overwrite(tok_e, sel_e_ref)
  tok_o = overwrite(tok_o, sel_o_ref)

  comb_e, comb_o = halves(comb_ref[...].reshape(_BBLK, _S2, 128))

  def layernorm(x):
    mu = jnp.mean(x, axis=2, keepdims=True)
    xc = x - jnp.broadcast_to(mu, (_BBLK, _S2, _D))
    var = jnp.mean(xc * xc, axis=2, keepdims=True)
    return (g_ref[...][None] * xc
            * jnp.broadcast_to(lax.rsqrt(var + 1e-5), (_BBLK, _S2, _D))
            + be_ref[...][None])

  xn_e = layernorm(tok_e + comb_e)
  xn_o = layernorm(tok_o + comb_o)

  pos_e, pos_o = halves(mm_tanh(pos_ref))
  neg_e, neg_o = halves(mm_tanh(neg_ref))

  def softplus(v):
    return jnp.maximum(v, 0.0) + jnp.log1p(jnp.exp(-jnp.abs(v)))

  def stats(xn, pos_t, neg_t, m_ref):
    pos_l = jnp.sum(xn * pos_t, axis=2, keepdims=True)
    neg_l = jnp.sum(xn * neg_t, axis=2, keepdims=True)
    m = m_ref[...][:, :, None].astype(f32)
    return (jnp.sum(m), jnp.sum(pos_l * m), jnp.sum(neg_l * m),
            jnp.sum(neg_l * neg_l * m),
            jnp.sum((softplus(-pos_l) + softplus(neg_l)) * m),
            jnp.max(jnp.where(m > 0, neg_l, -jnp.inf)))

  se = stats(xn_e, pos_e, neg_e, m_e_ref)
  so = stats(xn_o, pos_o, neg_o, m_o_ref)
  blk_m = se[0] + so[0]
  blk_p = se[1] + so[1]
  blk_n = se[2] + so[2]
  blk_n2 = se[3] + so[3]
  blk_l = se[4] + so[4]
  blk_mx = jnp.maximum(se[5], so[5])

  @pl.when(i == 0)
  def _():
    acc[0] = 0.0
    acc[1] = 0.0
    acc[2] = 0.0
    acc[3] = 0.0
    acc[4] = 0.0
    acc[5] = -jnp.inf

  acc[0] = acc[0] + blk_m
  acc[1] = acc[1] + blk_p
  acc[2] = acc[2] + blk_n
  acc[3] = acc[3] + blk_n2
  acc[4] = acc[4] + blk_l
  acc[5] = jnp.maximum(acc[5], blk_mx)

  @pl.when(i == _GRID - 1)
  def _():
    denom = acc[0] + 1e-8
    neg_score = acc[2] / denom
    out_ref[0] = acc[4] / denom
    out_ref[1] = acc[1] / denom
    out_ref[2] = neg_score
    out_ref[3] = (acc[3] - 2.0 * neg_score * acc[2]
                  + neg_score * neg_score * acc[0]) / denom
    out_ref[4] = acc[5]
    out_ref[5] = 0.0
    out_ref[6] = 0.0
    out_ref[7] = 0.0


def _tc_reduce(tok_g, pos_g, neg_g, comb_g, user_g,
               sel_e, sel_o, m_e, m_o,
               Wd_item, bd_item, W_user, b_user, ln_g, ln_b):
  f32 = jnp.float32
  val2 = pl.BlockSpec((_RB2, 128), lambda i: (i, 0))
  msk2 = pl.BlockSpec((_BBLK, _S2), lambda i: (i, 0))
  mat = pl.BlockSpec((128, 128), lambda i: (0, 0))
  vec = pl.BlockSpec((1, _D), lambda i: (0, 0))
  return pl.pallas_call(
      _tc_body,
      grid=(_GRID,),
      in_specs=[
          val2, val2, val2, val2,
          pl.BlockSpec((_BBLK, _D), lambda i: (i, 0)),
          msk2, msk2, msk2, msk2,
          mat, pl.BlockSpec((1, 128), lambda i: (0, 0)),
          pl.BlockSpec((_D, _D), lambda i: (0, 0)), vec, vec, vec,
      ],
      out_specs=pl.BlockSpec((8,), lambda i: (0,), memory_space=pltpu.SMEM),
      out_shape=jax.ShapeDtypeStruct((8,), f32),
      scratch_shapes=[pltpu.SMEM((8,), f32)],
  )(tok_g, pos_g, neg_g, comb_g, user_g, sel_e, sel_o, m_e, m_o,
    Wd_item, bd_item, W_user, b_user, ln_g, ln_b)


def kernel(user_id, j, user_feat, id_seq, feat_seq, pos_seq, pos_feat,
           neg_seq, neg_feat, inter_time, act_type, token_type,
           user_table, item_table, uf0, uf1, uf2, if0, if1,
           W_user, b_user, W_item, b_item, time_table, act_table, tok_table,
           ln_g, ln_b):
  i32 = jnp.int32
  f32 = jnp.float32

  def shard(a):
    return a.astype(i32).reshape(_NW, _NCH, _C)

  tok_id = shard(id_seq)
  tok_f0 = shard(feat_seq[..., 0])
  tok_f1 = shard(feat_seq[..., 1])
  pos_id = shard(pos_seq)
  pos_f0 = shard(pos_feat[..., 0])
  pos_f1 = shard(pos_feat[..., 1])
  neg_id = shard(neg_seq)
  neg_f0 = shard(neg_feat[..., 0])
  neg_f1 = shard(neg_feat[..., 1])
  comb_i = shard((inter_time % _NTIME) * (_NACT * _NTOK)
                 + act_type * _NTOK + token_type)
  uid = user_id.astype(i32).reshape(_NW, _UPW)
  u0 = user_feat[:, 0].astype(i32).reshape(_NW, _UPW)
  u1 = user_feat[:, 1].astype(i32).reshape(_NW, _UPW)
  u2 = user_feat[:, 2].astype(i32).reshape(_NW, _UPW)

  if0h = if0 * 0.5
  if1h = if1 * 0.5
  third = f32(1.0 / 3.0)
  uf0t = uf0 * third
  uf1t = uf1 * third
  uf2t = uf2 * third
  comb_t = (time_table[:, None, None, :] + act_table[None, :, None, :]
            + tok_table[None, None, :, :]).reshape(_NTIME * _NACT * _NTOK, _D)

  tok_g, pos_g, neg_g, comb_g, user_g = _sc_gather(
      item_table, if0h, if1h, comb_t, user_table, uf0t, uf1t, uf2t,
      tok_id, tok_f0, tok_f1, pos_id, pos_f0, pos_f1,
      neg_id, neg_f0, neg_f1, comb_i, uid, u0, u1, u2)

  jj = jnp.clip(j, 0, _S - 1).astype(i32)
  k = jnp.arange(_S2, dtype=i32)[None, :]
  sel_e = (2 * k == jj[:, None]).astype(i32)
  sel_o = (2 * k + 1 == jj[:, None]).astype(i32)
  tt = token_type.astype(i32)
  m_e = (tt[:, 0::2] == 1).astype(i32)
  m_o = (tt[:, 1::2] == 1).astype(i32)

  Wi = W_item.astype(f32)
  Wd = jnp.zeros((128, 128), f32)
  Wd = lax.dynamic_update_slice(Wd, Wi, (0, 0))
  Wd = lax.dynamic_update_slice(Wd, Wi, (_D, _D))
  bi = b_item.reshape(1, _D).astype(f32)
  bd = jnp.concatenate([bi, bi], axis=1)

  out = _tc_reduce(
      tok_g.reshape(_N // 2, 128), pos_g.reshape(_N // 2, 128),
      neg_g.reshape(_N // 2, 128), comb_g.reshape(_N // 2, 128),
      user_g, sel_e, sel_o, m_e, m_o,
      Wd, bd,
      W_user.astype(f32), b_user.reshape(1, _D).astype(f32),
      ln_g.reshape(1, _D).astype(f32), ln_b.reshape(1, _D).astype(f32))

  return (out[0], out[1], out[2], out[3], out[4])


# speedup vs baseline: 1.3306x; 1.0214x over previous
import functools

import jax
import jax.numpy as jnp
from jax import lax
from jax.experimental import pallas as pl
from jax.experimental.pallas import tpu as pltpu
from jax.experimental.pallas import tpu_sc as plsc

_B = 1024
_S = 200
_D = 64
_NTIME = 512
_NACT = 8
_NTOK = 4
_N = _B * _S
_NC = 2
_NS = 16
_NW = _NC * _NS
_NH = 2
_N2 = _N // _NH
_B2 = _B // _NH
_PW = _N2 // _NW
_UPW = _B2 // _NW

_BBLK = 32
_GRID = _B2 // _BBLK
_RB = _BBLK * _S


_C = 640
_NCH = _PW // _C


def _sc_gather_body(item_t, if0h, if1h, comb_t, user_t, uf0t, uf1t, uf2t,
                    tok_id, tok_f0, tok_f1,
                    pos_id, pos_f0, pos_f1,
                    neg_id, neg_f0, neg_f1,
                    comb_i, uid, u0, u1, u2,
                    tok_g, pos_g, neg_g, comb_g, user_g,
                    ia0, ia1, ia2, ib0, ib1, ib2, buf_a, buf_b, uidx,
                    sga, saa, swa, sgb, sab, swb):
  wid = lax.axis_index("s") * _NC + lax.axis_index("c")
  row0 = wid * _PW

  def start_g(tab, idxv, c, buf, sem, add=False):
    pltpu.async_copy(tab.at[idxv.at[c]], buf, sem, add=add)

  def wait_g(tab, idxv, c, buf, sem):
    pltpu.make_async_copy(tab.at[idxv.at[c]], buf, sem).wait()

  def start_w(buf, out, c, sem):
    pltpu.async_copy(buf, out.at[pl.ds(row0 + c * _C, _C)], sem)

  def wait_w(buf, out, c, sem):
    pltpu.make_async_copy(buf, out.at[pl.ds(row0 + c * _C, _C)], sem).wait()

  def phase(ta0, ta1, ta2, oa, tb0, tb1, tb2, ob):
    b_adds = tb1 is not None
    start_g(ta0, ia0, 0, buf_a, sga)
    start_g(tb0, ib0, 0, buf_b, sgb)

    def body(c, carry):
      wait_g(ta0, ia0, c, buf_a, sga)
      start_g(ta1, ia1, c, buf_a, saa, add=True)
      start_g(ta2, ia2, c, buf_a, saa, add=True)
      wait_g(tb0, ib0, c, buf_b, sgb)
      if b_adds:
        start_g(tb1, ib1, c, buf_b, sab, add=True)
        start_g(tb2, ib2, c, buf_b, sab, add=True)
      wait_g(ta1, ia1, c, buf_a, saa)
      wait_g(ta2, ia2, c, buf_a, saa)
      start_w(buf_a, oa, c, swa)
      if b_adds:
        wait_g(tb1, ib1, c, buf_b, sab)
        wait_g(tb2, ib2, c, buf_b, sab)
      start_w(buf_b, ob, c, swb)
      wait_w(buf_a, oa, c, swa)
      wait_w(buf_b, ob, c, swb)

      @pl.when(c + 1 < _NCH)
      def _():
        start_g(ta0, ia0, c + 1, buf_a, sga)
        start_g(tb0, ib0, c + 1, buf_b, sgb)

      return carry

    lax.fori_loop(0, _NCH, body, 0)

  pltpu.sync_copy(tok_id.at[wid], ia0)
  pltpu.sync_copy(tok_f0.at[wid], ia1)
  pltpu.sync_copy(tok_f1.at[wid], ia2)
  pltpu.sync_copy(pos_id.at[wid], ib0)
  pltpu.sync_copy(pos_f0.at[wid], ib1)
  pltpu.sync_copy(pos_f1.at[wid], ib2)
  phase(item_t, if0h, if1h, tok_g, item_t, if0h, if1h, pos_g)

  pltpu.sync_copy(neg_id.at[wid], ia0)
  pltpu.sync_copy(neg_f0.at[wid], ia1)
  pltpu.sync_copy(neg_f1.at[wid], ia2)
  pltpu.sync_copy(comb_i.at[wid], ib0)
  phase(item_t, if0h, if1h, neg_g, comb_t, None, None, comb_g)

  pltpu.sync_copy(uid.at[wid], uidx.at[0])
  pltpu.sync_copy(u0.at[wid], uidx.at[1])
  pltpu.sync_copy(u1.at[wid], uidx.at[2])
  pltpu.sync_copy(u2.at[wid], uidx.at[3])
  urows = buf_a.at[pl.ds(0, _UPW)]
  pltpu.async_copy(user_t.at[uidx.at[0]], urows, sga).wait()
  c1 = pltpu.async_copy(uf0t.at[uidx.at[1]], urows, sga, add=True)
  c2 = pltpu.async_copy(uf1t.at[uidx.at[2]], urows, saa, add=True)
  c3 = pltpu.async_copy(uf2t.at[uidx.at[3]], urows, swa, add=True)
  c1.wait()
  c2.wait()
  c3.wait()
  pltpu.sync_copy(urows, user_g.at[pl.ds(wid * _UPW, _UPW)])


def _sc_gather(item_t, if0h, if1h, comb_t, user_t, uf0t, uf1t, uf2t,
               tok_id, tok_f0, tok_f1, pos_id, pos_f0, pos_f1,
               neg_id, neg_f0, neg_f1, comb_i, uid, u0, u1, u2):
  mesh = plsc.VectorSubcoreMesh(core_axis_name="c", subcore_axis_name="s")
  f32 = jnp.float32
  out_type = (
      jax.ShapeDtypeStruct((_N2, _D), f32),
      jax.ShapeDtypeStruct((_N2, _D), f32),
      jax.ShapeDtypeStruct((_N2, _D), f32),
      jax.ShapeDtypeStruct((_N2, _D), f32),
      jax.ShapeDtypeStruct((_B2, _D), f32),
  )
  scratch = [
      pltpu.VMEM((_NCH, _C), jnp.int32),
      pltpu.VMEM((_NCH, _C), jnp.int32),
      pltpu.VMEM((_NCH, _C), jnp.int32),
      pltpu.VMEM((_NCH, _C), jnp.int32),
      pltpu.VMEM((_NCH, _C), jnp.int32),
      pltpu.VMEM((_NCH, _C), jnp.int32),
      pltpu.VMEM((_C, _D), f32),
      pltpu.VMEM((_C, _D), f32),
      pltpu.VMEM((4, _UPW), jnp.int32),
      pltpu.SemaphoreType.DMA,
      pltpu.SemaphoreType.DMA,
      pltpu.SemaphoreType.DMA,
      pltpu.SemaphoreType.DMA,
      pltpu.SemaphoreType.DMA,
      pltpu.SemaphoreType.DMA,
  ]
  run = pl.kernel(_sc_gather_body, out_type=out_type, mesh=mesh,
                  scratch_types=scratch,
                  compiler_params=pltpu.CompilerParams(
                      use_tc_tiling_on_sc=False))
  return run(item_t, if0h, if1h, comb_t, user_t, uf0t, uf1t, uf2t,
             tok_id, tok_f0, tok_f1, pos_id, pos_f0, pos_f1,
             neg_id, neg_f0, neg_f1, comb_i, uid, u0, u1, u2)


_S2 = _S // 2
_RB2 = _BBLK * _S2


def _tc_body(final, tok_ref, pos_ref, neg_ref, comb_ref, user_ref,
             sel_e_ref, sel_o_ref, m_e_ref, m_o_ref, prev_ref,
             wi_ref, bi_ref, wu_ref, bu_ref, g_ref, be_ref, out_ref, acc):
  i = pl.program_id(0)
  f32 = jnp.float32

  wi = wi_ref[...]
  bi = bi_ref[...]

  def mm_tanh(ref):
    out = jnp.tanh(jnp.dot(ref[...], wi, preferred_element_type=f32) + bi)
    return out.reshape(_BBLK, _S2, 128)

  def halves(v):
    return v[..., :_D], v[..., _D:]

  tok_e, tok_o = halves(mm_tanh(tok_ref))

  user_token = jnp.tanh(
      jnp.dot(user_ref[...], wu_ref[...], preferred_element_type=f32)
      + bu_ref[...])
  usr = jnp.broadcast_to(user_token[:, None, :], (_BBLK, _S2, _D))

  def overwrite(t, sel_ref_):
    sel = jnp.broadcast_to(sel_ref_[...][:, :, None], (_BBLK, _S2, _D))
    return jnp.where(sel > 0, usr, t)

  tok_e = overwrite(tok_e, sel_e_ref)
  tok_o = overwrite(tok_o, sel_o_ref)

  comb_e, comb_o = halves(comb_ref[...].reshape(_BBLK, _S2, 128))

  def layernorm(x):
    mu = jnp.mean(x, axis=2, keepdims=True)
    xc = x - jnp.broadcast_to(mu, (_BBLK, _S2, _D))
    var = jnp.mean(xc * xc, axis=2, keepdims=True)
    return (g_ref[...][None] * xc
            * jnp.broadcast_to(lax.rsqrt(var + 1e-5), (_BBLK, _S2, _D))
            + be_ref[...][None])

  xn_e = layernorm(tok_e + comb_e)
  xn_o = layernorm(tok_o + comb_o)

  pos_e, pos_o = halves(mm_tanh(pos_ref))
  neg_e, neg_o = halves(mm_tanh(neg_ref))

  def softplus(v):
    return jnp.maximum(v, 0.0) + jnp.log1p(jnp.exp(-jnp.abs(v)))

  def stats(xn, pos_t, neg_t, m_ref):
    pos_l = jnp.sum(xn * pos_t, axis=2, keepdims=True)
    neg_l = jnp.sum(xn * neg_t, axis=2, keepdims=True)
    m = m_ref[...][:, :, None].astype(f32)
    return (jnp.sum(m), jnp.sum(pos_l * m), jnp.sum(neg_l * m),
            jnp.sum(neg_l * neg_l * m),
            jnp.sum((softplus(-pos_l) + softplus(neg_l)) * m),
            jnp.max(jnp.where(m > 0, neg_l, -jnp.inf)))

  se = stats(xn_e, pos_e, neg_e, m_e_ref)
  so = stats(xn_o, pos_o, neg_o, m_o_ref)
  blk_m = se[0] + so[0]
  blk_p = se[1] + so[1]
  blk_n = se[2] + so[2]
  blk_n2 = se[3] + so[3]
  blk_l = se[4] + so[4]
  blk_mx = jnp.maximum(se[5], so[5])

  @pl.when(i == 0)
  def _():
    acc[0] = prev_ref[0]
    acc[1] = prev_ref[1]
    acc[2] = prev_ref[2]
    acc[3] = prev_ref[3]
    acc[4] = prev_ref[4]
    acc[5] = prev_ref[5]

  acc[0] = acc[0] + blk_m
  acc[1] = acc[1] + blk_p
  acc[2] = acc[2] + blk_n
  acc[3] = acc[3] + blk_n2
  acc[4] = acc[4] + blk_l
  acc[5] = jnp.maximum(acc[5], blk_mx)

  @pl.when(i == _GRID - 1)
  def _():
    if final:
      denom = acc[0] + 1e-8
      neg_score = acc[2] / denom
      out_ref[0] = acc[4] / denom
      out_ref[1] = acc[1] / denom
      out_ref[2] = neg_score
      out_ref[3] = (acc[3] - 2.0 * neg_score * acc[2]
                    + neg_score * neg_score * acc[0]) / denom
      out_ref[4] = acc[5]
      out_ref[5] = 0.0
    else:
      out_ref[0] = acc[0]
      out_ref[1] = acc[1]
      out_ref[2] = acc[2]
      out_ref[3] = acc[3]
      out_ref[4] = acc[4]
      out_ref[5] = acc[5]
    out_ref[6] = 0.0
    out_ref[7] = 0.0


def _tc_reduce(final, tok_g, pos_g, neg_g, comb_g, user_g,
               sel_e, sel_o, m_e, m_o, prev,
               Wd_item, bd_item, W_user, b_user, ln_g, ln_b):
  f32 = jnp.float32
  val2 = pl.BlockSpec((_RB2, 128), lambda i: (i, 0))
  msk2 = pl.BlockSpec((_BBLK, _S2), lambda i: (i, 0))
  mat = pl.BlockSpec((128, 128), lambda i: (0, 0))
  vec = pl.BlockSpec((1, _D), lambda i: (0, 0))
  return pl.pallas_call(
      functools.partial(_tc_body, final),
      grid=(_GRID,),
      in_specs=[
          val2, val2, val2, val2,
          pl.BlockSpec((_BBLK, _D), lambda i: (i, 0)),
          msk2, msk2, msk2, msk2,
          pl.BlockSpec((8,), lambda i: (0,), memory_space=pltpu.SMEM),
          mat, pl.BlockSpec((1, 128), lambda i: (0, 0)),
          pl.BlockSpec((_D, _D), lambda i: (0, 0)), vec, vec, vec,
      ],
      out_specs=pl.BlockSpec((8,), lambda i: (0,), memory_space=pltpu.SMEM),
      out_shape=jax.ShapeDtypeStruct((8,), f32),
      scratch_shapes=[pltpu.SMEM((8,), f32)],
  )(tok_g, pos_g, neg_g, comb_g, user_g, sel_e, sel_o, m_e, m_o, prev,
    Wd_item, bd_item, W_user, b_user, ln_g, ln_b)


def kernel(user_id, j, user_feat, id_seq, feat_seq, pos_seq, pos_feat,
           neg_seq, neg_feat, inter_time, act_type, token_type,
           user_table, item_table, uf0, uf1, uf2, if0, if1,
           W_user, b_user, W_item, b_item, time_table, act_table, tok_table,
           ln_g, ln_b):
  i32 = jnp.int32
  f32 = jnp.float32

  def shard(a):
    return a.astype(i32).reshape(_NH, _NW, _NCH, _C)

  tok_id = shard(id_seq)
  tok_f0 = shard(feat_seq[..., 0])
  tok_f1 = shard(feat_seq[..., 1])
  pos_id = shard(pos_seq)
  pos_f0 = shard(pos_feat[..., 0])
  pos_f1 = shard(pos_feat[..., 1])
  neg_id = shard(neg_seq)
  neg_f0 = shard(neg_feat[..., 0])
  neg_f1 = shard(neg_feat[..., 1])
  comb_i = shard((inter_time % _NTIME) * (_NACT * _NTOK)
                 + act_type * _NTOK + token_type)
  uid = user_id.astype(i32).reshape(_NH, _NW, _UPW)
  u0 = user_feat[:, 0].astype(i32).reshape(_NH, _NW, _UPW)
  u1 = user_feat[:, 1].astype(i32).reshape(_NH, _NW, _UPW)
  u2 = user_feat[:, 2].astype(i32).reshape(_NH, _NW, _UPW)

  if0h = if0 * 0.5
  if1h = if1 * 0.5
  third = f32(1.0 / 3.0)
  uf0t = uf0 * third
  uf1t = uf1 * third
  uf2t = uf2 * third
  comb_t = (time_table[:, None, None, :] + act_table[None, :, None, :]
            + tok_table[None, None, :, :]).reshape(_NTIME * _NACT * _NTOK, _D)

  gathered = [
      _sc_gather(
          item_table, if0h, if1h, comb_t, user_table, uf0t, uf1t, uf2t,
          tok_id[h], tok_f0[h], tok_f1[h], pos_id[h], pos_f0[h], pos_f1[h],
          neg_id[h], neg_f0[h], neg_f1[h], comb_i[h],
          uid[h], u0[h], u1[h], u2[h])
      for h in range(_NH)
  ]

  jj = jnp.clip(j, 0, _S - 1).astype(i32)
  k = jnp.arange(_S2, dtype=i32)[None, :]
  sel_e = (2 * k == jj[:, None]).astype(i32)
  sel_o = (2 * k + 1 == jj[:, None]).astype(i32)
  tt = token_type.astype(i32)
  m_e = (tt[:, 0::2] == 1).astype(i32)
  m_o = (tt[:, 1::2] == 1).astype(i32)

  Wi = W_item.astype(f32)
  Wd = jnp.zeros((128, 128), f32)
  Wd = lax.dynamic_update_slice(Wd, Wi, (0, 0))
  Wd = lax.dynamic_update_slice(Wd, Wi, (_D, _D))
  bi = b_item.reshape(1, _D).astype(f32)
  bd = jnp.concatenate([bi, bi], axis=1)
  Wu = W_user.astype(f32)
  bu = b_user.reshape(1, _D).astype(f32)
  g2 = ln_g.reshape(1, _D).astype(f32)
  b2 = ln_b.reshape(1, _D).astype(f32)

  carry = jnp.array([0, 0, 0, 0, 0, -jnp.inf, 0, 0], f32)
  for h in range(_NH):
    tok_g, pos_g, neg_g, comb_g, user_g = gathered[h]
    b0, b1 = h * _B2, (h + 1) * _B2
    carry = _tc_reduce(
        h == _NH - 1,
        tok_g.reshape(_N2 // 2, 128), pos_g.reshape(_N2 // 2, 128),
        neg_g.reshape(_N2 // 2, 128), comb_g.reshape(_N2 // 2, 128),
        user_g, sel_e[b0:b1], sel_o[b0:b1], m_e[b0:b1], m_o[b0:b1],
        carry, Wd, bd, Wu, bu, g2, b2)

  return (carry[0], carry[1], carry[2], carry[3], carry[4])
